# Initial kernel scaffold; baseline (speedup 1.0000x reference)
import jax
import jax.numpy as jnp
from jax.experimental import pallas as pl


def kernel(scores):
    raise NotImplementedError("write your pallas kernel here")


# speedup vs baseline: 139.1746x; 139.1746x over previous
import jax
import jax.numpy as jnp
from jax.experimental import pallas as pl

_ROWS = 128
_COLS = 32768
_R_BLK = 16


def _argmax_rank_kernel(s_ref, o_ref):
    s = s_ref[:, :]
    pivot = s[:, 0:1]
    cnt = jnp.sum((s > pivot).astype(jnp.int32), axis=1, keepdims=True)
    iota = jax.lax.broadcasted_iota(jnp.int32, (_R_BLK, _COLS), 1)
    o_ref[:, :] = (iota == cnt).astype(jnp.float32)


def kernel(scores):
    return pl.pallas_call(
        _argmax_rank_kernel,
        out_shape=jax.ShapeDtypeStruct((_ROWS, _COLS), jnp.float32),
        grid=(_ROWS // _R_BLK,),
        in_specs=[pl.BlockSpec((_R_BLK, _COLS), lambda i: (i, 0))],
        out_specs=pl.BlockSpec((_R_BLK, _COLS), lambda i: (i, 0)),
    )(scores)
